# baseline (speedup 1.0000x reference)
import functools

import jax
import jax.numpy as jnp
from jax import lax
from jax.experimental import pallas as pl
from jax.experimental.pallas import tpu as pltpu
from jax.experimental.pallas import tpu_sc as plsc

N_NODES = 10000
N_HYPEREDGES = 5000
NNZ = 320000
E_INNER = 640000
D = 128
NUM_CLASSES = 10
NUM_GRAPHS = 16
NUM_LAYERS = 2
INNER_LAYERS = 2

CH = 6400
NCHUNK = NNZ // CH
EB = 128
RPT = CH // 16


SB = 512
NSUB = SB // EB


def _agg_body(x_hbm, srcs_hbm, dsts_hbm, offs2_hbm, out_hbm,
              acc_sh, offv, idxw, dstw, gidx, sidx, rows, semg, sems):
    cid = lax.axis_index("c")
    sid = lax.axis_index("s")
    iota = lax.iota(jnp.int32, 16)

    def chunk_body(ci, chunk_carry):
        c = cid * (NCHUNK // 2) + ci
        base = pl.multiple_of(c * CH, CH)
        r0 = pl.multiple_of(base + sid * RPT, 8)
        pltpu.sync_copy(x_hbm.at[pl.ds(r0, RPT)],
                        acc_sh.at[pl.ds(sid * RPT, RPT)])
        pltpu.sync_copy(offs2_hbm.at[c], offv)
        plsc.subcore_barrier()
        v = offv[...]
        lo = v[0]
        hi = v[1]
        per_tile = lax.div(hi - lo + 15, 16)
        e0 = lo + sid * per_tile
        e1 = jnp.minimum(e0 + per_tile, hi)
        nsb = jnp.maximum(lax.div(e1 - e0 + (SB - 1), SB), 0)

        def super_body(b, carry):
            s0 = e0 + b * SB
            sa = (s0 // 8) * 8
            sh = s0 - sa
            pltpu.sync_copy(srcs_hbm.at[pl.ds(sa, SB + 16)], idxw)
            pltpu.sync_copy(dsts_hbm.at[pl.ds(sa, SB + 16)], dstw)
            for j in range(NSUB):
                for j16 in range(EB // 16):
                    o = j * EB + j16 * 16
                    pos = s0 + o + iota
                    sv = idxw[pl.ds(sh + o, 16)]
                    dv = dstw[pl.ds(sh + o, 16)]
                    valid = pos < e1
                    gidx[j, pl.ds(j16 * 16, 16)] = jnp.where(valid, sv, 0)
                    sidx[j, pl.ds(j16 * 16, 16)] = jnp.where(valid, dv - base, CH)
            gh = [pltpu.async_copy(x_hbm.at[gidx.at[j]], rows.at[j], semg)
                  for j in range(NSUB)]
            for h in gh:
                h.wait()
            for j in range(NSUB):
                pltpu.sync_copy(rows.at[j], acc_sh.at[sidx.at[j]], add=True)
            return carry

        lax.fori_loop(0, nsb, super_body, 0)
        plsc.subcore_barrier()
        pltpu.sync_copy(acc_sh.at[pl.ds(sid * RPT, RPT)],
                        out_hbm.at[pl.ds(r0, RPT)])
        return chunk_carry

    lax.fori_loop(0, NCHUNK // 2, chunk_body, 0)


_sc_agg = pl.kernel(
    _agg_body,
    out_type=jax.ShapeDtypeStruct((NNZ, D), jnp.float32),
    mesh=plsc.VectorSubcoreMesh(core_axis_name="c", subcore_axis_name="s"),
    scratch_types=[
        pltpu.VMEM_SHARED((CH + 8, D), jnp.float32),
        pltpu.VMEM((16,), jnp.int32),
        pltpu.VMEM((SB + 16,), jnp.int32),
        pltpu.VMEM((SB + 16,), jnp.int32),
        pltpu.VMEM((NSUB, EB), jnp.int32),
        pltpu.VMEM((NSUB, EB), jnp.int32),
        pltpu.VMEM((NSUB, EB, D), jnp.float32),
        pltpu.SemaphoreType.DMA,
        pltpu.SemaphoreType.DMA,
    ],
)


def _edge_prep(edge_index):
    src, dst = edge_index[0], edge_index[1]
    order = jnp.argsort(dst)
    src_s = jnp.take(src, order)
    dst_s = jnp.take(dst, order)
    bounds = jnp.arange(0, NNZ + 1, CH, dtype=jnp.int32)
    offs = jnp.searchsorted(dst_s, bounds).astype(jnp.int32)
    offs2 = jnp.tile(jnp.stack([offs[:-1], offs[1:]], axis=1), (1, 8))
    src_p = jnp.concatenate([src_s, jnp.zeros((256,), jnp.int32)])
    dst_p = jnp.concatenate([dst_s, jnp.zeros((256,), jnp.int32)])
    return src_p, dst_p, offs2


def _mlp_body(h_ref, w1_ref, b1_ref, w2_ref, b2_ref, o_ref):
    h = jnp.maximum(jnp.dot(h_ref[...], w1_ref[...],
                            preferred_element_type=jnp.float32)
                    + b1_ref[...], 0.0)
    o_ref[...] = jnp.maximum(jnp.dot(h, w2_ref[...],
                                     preferred_element_type=jnp.float32)
                             + b2_ref[...], 0.0)


def _gin_mlp(h, w1, b1, w2, b2):
    n = h.shape[0]
    blk = 2000
    row_spec = pl.BlockSpec((blk, D), lambda i: (i, 0))
    w_spec = pl.BlockSpec((D, D), lambda i: (0, 0))
    b_spec = pl.BlockSpec((1, D), lambda i: (0, 0))
    return pl.pallas_call(
        _mlp_body,
        grid=(n // blk,),
        in_specs=[row_spec, w_spec, b_spec, w_spec, b_spec],
        out_specs=row_spec,
        out_shape=jax.ShapeDtypeStruct((n, D), jnp.float32),
    )(h, w1, b1.reshape(1, D), w2, b2.reshape(1, D))


def _emb_body(x_ref, w_ref, b_ref, o_ref):
    o_ref[...] = jnp.dot(x_ref[...], w_ref[...],
                         preferred_element_type=jnp.float32) + b_ref[...]


def _emb(x, w, b):
    n = x.shape[0]
    blk = 2000
    return pl.pallas_call(
        _emb_body,
        grid=(n // blk,),
        in_specs=[pl.BlockSpec((blk, D), lambda i: (i, 0)),
                  pl.BlockSpec((D, D), lambda i: (0, 0)),
                  pl.BlockSpec((1, D), lambda i: (0, 0))],
        out_specs=pl.BlockSpec((blk, D), lambda i: (i, 0)),
        out_shape=jax.ShapeDtypeStruct((n, D), jnp.float32),
    )(x, w, b.reshape(1, D))


def kernel(x_N, W_emb, b_emb, gin_W1, gin_b1, gin_W2, gin_b2, W_pred, b_pred,
           ori_node_idx, node2edge, ori_edge_idx, edge2node,
           edge_index_N, edge_index_E, batch):
    node_x = _emb(x_N, W_emb, b_emb)
    prep_N = _edge_prep(edge_index_N)
    prep_E = _edge_prep(edge_index_E)
    xs = [node_x]
    for l in range(NUM_LAYERS):
        _nx = node_x[ori_node_idx]
        for c in range(INNER_LAYERS):
            idx = l * 4 + c
            h = _sc_agg(_nx, *prep_N)
            _nx = _gin_mlp(h, gin_W1[idx], gin_b1[idx],
                           gin_W2[idx], gin_b2[idx])
        edge_x = jax.nn.relu(jax.ops.segment_sum(_nx, node2edge,
                                                 num_segments=N_HYPEREDGES))
        _ex = edge_x[ori_edge_idx]
        for c in range(INNER_LAYERS):
            idx = l * 4 + 2 + c
            h = _sc_agg(_ex, *prep_E)
            _ex = _gin_mlp(h, gin_W1[idx], gin_b1[idx],
                           gin_W2[idx], gin_b2[idx])
        node_x = jax.nn.relu(jax.ops.segment_sum(_ex, edge2node,
                                                 num_segments=N_NODES))
        xs.append(node_x)
    score = jnp.zeros((NUM_GRAPHS, NUM_CLASSES), jnp.float32)
    for i, x in enumerate(xs):
        pooled = jax.ops.segment_sum(x[ori_node_idx], batch,
                                     num_segments=NUM_GRAPHS)
        score = score + pooled @ W_pred[i] + b_pred[i]
    return score


# speedup vs baseline: 1.5356x; 1.5356x over previous
import functools

import jax
import jax.numpy as jnp
from jax import lax
from jax.experimental import pallas as pl
from jax.experimental.pallas import tpu as pltpu
from jax.experimental.pallas import tpu_sc as plsc

N_NODES = 10000
N_HYPEREDGES = 5000
NNZ = 320000
E_INNER = 640000
D = 128
NUM_CLASSES = 10
NUM_GRAPHS = 16
NUM_LAYERS = 2
INNER_LAYERS = 2

CH = 6400
NCHUNK = NNZ // CH
EB = 128
RPT = CH // 16


SB = 512
NSUB = SB // EB


def _agg_body(x_hbm, srcs_hbm, dstl_hbm, offs2_hbm, out_hbm,
              acc_sh, offv, widx, wdst, gidxm, sidxm,
              ib0, db0, rb0, ib1, db1, rb1, ib2, db2, rb2, ib3, db3, rb3,
              semw, semg, sems):
    IB = [ib0, ib1, ib2, ib3]
    DB = [db0, db1, db2, db3]
    RB = [rb0, rb1, rb2, rb3]
    cid = lax.axis_index("c")
    sid = lax.axis_index("s")
    iota = lax.iota(jnp.int32, 16)

    def chunk_body(ci, chunk_carry):
        c = cid * (NCHUNK // 2) + ci
        base = pl.multiple_of(c * CH, CH)
        row0 = pl.multiple_of(base + sid * RPT, 8)
        pltpu.sync_copy(x_hbm.at[pl.ds(row0, RPT)],
                        acc_sh.at[pl.ds(sid * RPT, RPT)])
        pltpu.sync_copy(offs2_hbm.at[c], offv)
        plsc.subcore_barrier()
        v = offv[...]
        lo = v[0]
        hi = v[1]
        per_tile = lax.div(hi - lo + 15, 16)
        e0 = lo + sid * per_tile
        e1 = jnp.minimum(e0 + per_tile, hi)
        e1 = jnp.maximum(e1, e0)
        core_start = lax.div(e0 + 127, EB) * EB
        core_end = jnp.maximum(lax.div(e1, EB) * EB, core_start)
        lead_end = jnp.minimum(core_start, e1)
        tail_start = jnp.maximum(core_end, lead_end)
        nb = lax.div(core_end - core_start, EB)

        def super_body(g, carry):
            s_base = core_start + g * (4 * EB)
            bb = [pl.multiple_of(s_base + j * EB, 8) for j in range(4)]
            for j in range(4):
                @pl.when(bb[j] < core_end)
                def _(j=j):
                    pltpu.async_copy(srcs_hbm.at[pl.ds(bb[j], EB)], IB[j], semw)
                    pltpu.async_copy(dstl_hbm.at[pl.ds(bb[j], EB)], DB[j], semw)
            for j in range(4):
                @pl.when(bb[j] < core_end)
                def _(j=j):
                    pltpu.make_async_copy(srcs_hbm.at[pl.ds(bb[j], EB)], IB[j], semw).wait()
                    pltpu.make_async_copy(dstl_hbm.at[pl.ds(bb[j], EB)], DB[j], semw).wait()
                    pltpu.async_copy(x_hbm.at[IB[j]], RB[j], semg)
            for j in range(4):
                @pl.when(bb[j] < core_end)
                def _(j=j):
                    pltpu.make_async_copy(x_hbm.at[IB[j]], RB[j], semg).wait()
                    pltpu.async_copy(RB[j], acc_sh.at[DB[j]], sems, add=True)
            for j in range(4):
                @pl.when(bb[j] < core_end)
                def _(j=j):
                    pltpu.make_async_copy(RB[j], acc_sh.at[DB[j]], sems).wait()
            return carry

        lax.fori_loop(0, lax.div(nb + 3, 4), super_body, 0)

        def masked_batch(mstart, mend):
            wa = lax.div(mstart, 8) * 8
            shft = mstart - wa
            pltpu.sync_copy(srcs_hbm.at[pl.ds(wa, EB + 8)], widx)
            pltpu.sync_copy(dstl_hbm.at[pl.ds(wa, EB + 8)], wdst)
            for i in range(EB // 16):
                pos = mstart + i * 16 + iota
                sv = widx[pl.ds(shft + i * 16, 16)]
                dv = wdst[pl.ds(shft + i * 16, 16)]
                ok = pos < mend
                gidxm[pl.ds(i * 16, 16)] = jnp.where(ok, sv, 0)
                sidxm[pl.ds(i * 16, 16)] = jnp.where(ok, dv, CH)
            pltpu.async_copy(x_hbm.at[gidxm], rb0, semg).wait()
            pltpu.sync_copy(rb0, acc_sh.at[sidxm], add=True)

        @pl.when(lead_end > e0)
        def _():
            masked_batch(e0, lead_end)

        @pl.when(e1 > tail_start)
        def _():
            masked_batch(tail_start, e1)

        plsc.subcore_barrier()
        pltpu.sync_copy(acc_sh.at[pl.ds(sid * RPT, RPT)],
                        out_hbm.at[pl.ds(row0, RPT)])
        return chunk_carry

    lax.fori_loop(0, NCHUNK // 2, chunk_body, 0)


_sc_agg = pl.kernel(
    _agg_body,
    out_type=jax.ShapeDtypeStruct((NNZ, D), jnp.float32),
    mesh=plsc.VectorSubcoreMesh(core_axis_name="c", subcore_axis_name="s"),
    scratch_types=[
        pltpu.VMEM_SHARED((CH + 8, D), jnp.float32),
        pltpu.VMEM((16,), jnp.int32),
        pltpu.VMEM((EB + 8,), jnp.int32),
        pltpu.VMEM((EB + 8,), jnp.int32),
        pltpu.VMEM((EB,), jnp.int32),
        pltpu.VMEM((EB,), jnp.int32),
    ] + [
        s for _ in range(4)
        for s in (pltpu.VMEM((EB,), jnp.int32),
                  pltpu.VMEM((EB,), jnp.int32),
                  pltpu.VMEM((EB, D), jnp.float32))
    ] + [
        pltpu.SemaphoreType.DMA,
        pltpu.SemaphoreType.DMA,
        pltpu.SemaphoreType.DMA,
    ],
)


def _edge_prep(edge_index):
    src, dst = edge_index[0], edge_index[1]
    order = jnp.argsort(dst)
    src_s = jnp.take(src, order)
    dst_s = jnp.take(dst, order)
    bounds = jnp.arange(0, NNZ + 1, CH, dtype=jnp.int32)
    offs = jnp.searchsorted(dst_s, bounds).astype(jnp.int32)
    offs2 = jnp.tile(jnp.stack([offs[:-1], offs[1:]], axis=1), (1, 8))
    src_p = jnp.concatenate([src_s, jnp.zeros((SB + 256,), jnp.int32)])
    dstl_p = jnp.concatenate([dst_s % CH, jnp.zeros((SB + 256,), jnp.int32)])
    return src_p, dstl_p, offs2


def _mlp_body(h_ref, w1_ref, b1_ref, w2_ref, b2_ref, o_ref):
    h = jnp.maximum(jnp.dot(h_ref[...], w1_ref[...],
                            preferred_element_type=jnp.float32)
                    + b1_ref[...], 0.0)
    o_ref[...] = jnp.maximum(jnp.dot(h, w2_ref[...],
                                     preferred_element_type=jnp.float32)
                             + b2_ref[...], 0.0)


def _gin_mlp(h, w1, b1, w2, b2):
    n = h.shape[0]
    blk = 2000
    row_spec = pl.BlockSpec((blk, D), lambda i: (i, 0))
    w_spec = pl.BlockSpec((D, D), lambda i: (0, 0))
    b_spec = pl.BlockSpec((1, D), lambda i: (0, 0))
    return pl.pallas_call(
        _mlp_body,
        grid=(n // blk,),
        in_specs=[row_spec, w_spec, b_spec, w_spec, b_spec],
        out_specs=row_spec,
        out_shape=jax.ShapeDtypeStruct((n, D), jnp.float32),
    )(h, w1, b1.reshape(1, D), w2, b2.reshape(1, D))


def _emb_body(x_ref, w_ref, b_ref, o_ref):
    o_ref[...] = jnp.dot(x_ref[...], w_ref[...],
                         preferred_element_type=jnp.float32) + b_ref[...]


def _emb(x, w, b):
    n = x.shape[0]
    blk = 2000
    return pl.pallas_call(
        _emb_body,
        grid=(n // blk,),
        in_specs=[pl.BlockSpec((blk, D), lambda i: (i, 0)),
                  pl.BlockSpec((D, D), lambda i: (0, 0)),
                  pl.BlockSpec((1, D), lambda i: (0, 0))],
        out_specs=pl.BlockSpec((blk, D), lambda i: (i, 0)),
        out_shape=jax.ShapeDtypeStruct((n, D), jnp.float32),
    )(x, w, b.reshape(1, D))


def kernel(x_N, W_emb, b_emb, gin_W1, gin_b1, gin_W2, gin_b2, W_pred, b_pred,
           ori_node_idx, node2edge, ori_edge_idx, edge2node,
           edge_index_N, edge_index_E, batch):
    node_x = _emb(x_N, W_emb, b_emb)
    prep_N = _edge_prep(edge_index_N)
    prep_E = _edge_prep(edge_index_E)
    xs = [node_x]
    for l in range(NUM_LAYERS):
        _nx = node_x[ori_node_idx]
        for c in range(INNER_LAYERS):
            idx = l * 4 + c
            h = _sc_agg(_nx, *prep_N)
            _nx = _gin_mlp(h, gin_W1[idx], gin_b1[idx],
                           gin_W2[idx], gin_b2[idx])
        edge_x = jax.nn.relu(jax.ops.segment_sum(_nx, node2edge,
                                                 num_segments=N_HYPEREDGES))
        _ex = edge_x[ori_edge_idx]
        for c in range(INNER_LAYERS):
            idx = l * 4 + 2 + c
            h = _sc_agg(_ex, *prep_E)
            _ex = _gin_mlp(h, gin_W1[idx], gin_b1[idx],
                           gin_W2[idx], gin_b2[idx])
        node_x = jax.nn.relu(jax.ops.segment_sum(_ex, edge2node,
                                                 num_segments=N_NODES))
        xs.append(node_x)
    score = jnp.zeros((NUM_GRAPHS, NUM_CLASSES), jnp.float32)
    for i, x in enumerate(xs):
        pooled = jax.ops.segment_sum(x[ori_node_idx], batch,
                                     num_segments=NUM_GRAPHS)
        score = score + pooled @ W_pred[i] + b_pred[i]
    return score


# speedup vs baseline: 3.2990x; 2.1484x over previous
import functools

import jax
import jax.numpy as jnp
from jax import lax
from jax.experimental import pallas as pl
from jax.experimental.pallas import tpu as pltpu
from jax.experimental.pallas import tpu_sc as plsc

N_NODES = 10000
N_HYPEREDGES = 5000
NNZ = 320000
E_INNER = 640000
D = 128
NUM_CLASSES = 10
NUM_GRAPHS = 16
NUM_LAYERS = 2
INNER_LAYERS = 2

NT = 32
ROWS_PT = NNZ // NT
W = 512
NWIN = -(-ROWS_PT // W)
EB = 128
WIN = EB + 16


def _agg_body(x_hbm, srcs_hbm, dsts_hbm, tbl_hbm, out_hbm,
              outb, offv, widxA, wdstA, rowsA, widxB, wdstB, rowsB,
              semwA, semwB, semgA, semgB):
    cid = lax.axis_index("c")
    sid = lax.axis_index("s")
    wid = cid * 16 + sid
    iota = lax.iota(jnp.int32, 16)

    def fire_windows(b, eb_a, widx, wdst, semw):
        wa = eb_a + b * EB
        pltpu.async_copy(srcs_hbm.at[pl.ds(wa, WIN)], widx, semw)
        pltpu.async_copy(dsts_hbm.at[pl.ds(wa, WIN)], wdst, semw)

    def wait_windows(b, eb_a, widx, wdst, semw):
        wa = eb_a + b * EB
        pltpu.make_async_copy(srcs_hbm.at[pl.ds(wa, WIN)], widx, semw).wait()
        pltpu.make_async_copy(dsts_hbm.at[pl.ds(wa, WIN)], wdst, semw).wait()

    def fire_gather(widx, rows, semg):
        pltpu.async_copy(x_hbm.at[widx.at[pl.ds(0, EB)]],
                         rows.at[pl.ds(0, EB)], semg)
        pltpu.async_copy(x_hbm.at[widx.at[pl.ds(EB, 16)]],
                         rows.at[pl.ds(EB, 16)], semg)

    def wait_gather(widx, rows, semg):
        pltpu.make_async_copy(x_hbm.at[widx.at[pl.ds(0, EB)]],
                              rows.at[pl.ds(0, EB)], semg).wait()
        pltpu.make_async_copy(x_hbm.at[widx.at[pl.ds(EB, 16)]],
                              rows.at[pl.ds(EB, 16)], semg).wait()

    def rmw(b, eb, ee, eb_a, wr0, wdst, rows):
        wa = eb_a + b * EB
        bstart = eb + b * EB
        bend = jnp.minimum(bstart + EB, ee)

        def group(g, carry):
            pos = wa + g * 16 + iota
            dvv = wdst[pl.ds(g * 16, 16)]
            ok = (pos >= bstart) & (pos < bend)
            dvl = jnp.where(ok, dvv - wr0, W)
            for l in range(16):
                dv = dvl[l]
                e = g * 16 + l
                for k in range(D // 16):
                    outb[dv, pl.ds(k * 16, 16)] = (
                        outb[dv, pl.ds(k * 16, 16)]
                        + rows[e, pl.ds(k * 16, 16)])
            return carry

        lax.fori_loop(0, WIN // 16, group, 0)

    def win_body(w, carry):
        pltpu.sync_copy(tbl_hbm.at[wid * NWIN + w], offv)
        v = offv[...]
        eb = v[0]
        ee = v[1]
        wr0 = pl.multiple_of(v[2], 8)
        pltpu.sync_copy(x_hbm.at[pl.ds(wr0, W)], outb.at[pl.ds(0, W)])
        eb_a = pl.multiple_of(lax.div(eb, 8) * 8, 8)
        nb = lax.div(ee - eb + (EB - 1), EB)

        @pl.when(nb > 0)
        def _():
            fire_windows(0, eb_a, widxA, wdstA, semwA)
            wait_windows(0, eb_a, widxA, wdstA, semwA)
            fire_gather(widxA, rowsA, semgA)

            @pl.when(nb > 1)
            def _():
                fire_windows(1, eb_a, widxB, wdstB, semwB)

            def pair_body(i, c2):
                bA = 2 * i
                bB = 2 * i + 1

                @pl.when(bB < nb)
                def _():
                    wait_windows(bB, eb_a, widxB, wdstB, semwB)
                    fire_gather(widxB, rowsB, semgB)
                wait_gather(widxA, rowsA, semgA)
                rmw(bA, eb, ee, eb_a, wr0, wdstA, rowsA)

                @pl.when(bB + 1 < nb)
                def _():
                    fire_windows(bB + 1, eb_a, widxA, wdstA, semwA)

                @pl.when(bB < nb)
                def _():
                    wait_gather(widxB, rowsB, semgB)
                    rmw(bB, eb, ee, eb_a, wr0, wdstB, rowsB)

                @pl.when(bB + 1 < nb)
                def _():
                    wait_windows(bB + 1, eb_a, widxA, wdstA, semwA)
                    fire_gather(widxA, rowsA, semgA)

                @pl.when(bB + 2 < nb)
                def _():
                    fire_windows(bB + 2, eb_a, widxB, wdstB, semwB)
                return c2

            lax.fori_loop(0, lax.div(nb + 1, 2), pair_body, 0)

        pltpu.sync_copy(outb.at[pl.ds(0, W)], out_hbm.at[pl.ds(wr0, W)])
        return carry

    lax.fori_loop(0, NWIN, win_body, 0)


_sc_agg = pl.kernel(
    _agg_body,
    out_type=jax.ShapeDtypeStruct((NNZ, D), jnp.float32),
    mesh=plsc.VectorSubcoreMesh(core_axis_name="c", subcore_axis_name="s"),
    scratch_types=[
        pltpu.VMEM((W + 8, D), jnp.float32),
        pltpu.VMEM((16,), jnp.int32),
        pltpu.VMEM((WIN,), jnp.int32),
        pltpu.VMEM((WIN,), jnp.int32),
        pltpu.VMEM((WIN, D), jnp.float32),
        pltpu.VMEM((WIN,), jnp.int32),
        pltpu.VMEM((WIN,), jnp.int32),
        pltpu.VMEM((WIN, D), jnp.float32),
        pltpu.SemaphoreType.DMA,
        pltpu.SemaphoreType.DMA,
        pltpu.SemaphoreType.DMA,
        pltpu.SemaphoreType.DMA,
    ],
)


def _edge_prep(edge_index):
    src, dst = edge_index[0], edge_index[1]
    order = jnp.argsort(dst)
    src_s = jnp.take(src, order)
    dst_s = jnp.take(dst, order)
    wr0_rel = jnp.minimum(jnp.arange(NWIN, dtype=jnp.int32) * W, ROWS_PT - W)
    wr0 = (jnp.arange(NT, dtype=jnp.int32)[:, None] * ROWS_PT
           + wr0_rel[None, :]).reshape(-1)
    bnds = jnp.stack([wr0, wr0 + W], 1).reshape(-1)
    ebee = jnp.searchsorted(dst_s, bnds).astype(jnp.int32).reshape(-1, 2)
    tbl = jnp.concatenate(
        [ebee, wr0[:, None], jnp.zeros((NT * NWIN, 1), jnp.int32)], axis=1)
    tbl16 = jnp.tile(tbl, (1, 4))
    src_p = jnp.concatenate([src_s, jnp.zeros((2 * WIN + 8,), jnp.int32)])
    dst_p = jnp.concatenate([dst_s, jnp.full((2 * WIN + 8,), NNZ, jnp.int32)])
    return src_p, dst_p, tbl16


def _mlp_body(h_ref, w1_ref, b1_ref, w2_ref, b2_ref, o_ref):
    h = jnp.maximum(jnp.dot(h_ref[...], w1_ref[...],
                            preferred_element_type=jnp.float32)
                    + b1_ref[...], 0.0)
    o_ref[...] = jnp.maximum(jnp.dot(h, w2_ref[...],
                                     preferred_element_type=jnp.float32)
                             + b2_ref[...], 0.0)


def _gin_mlp(h, w1, b1, w2, b2):
    n = h.shape[0]
    blk = 2000
    row_spec = pl.BlockSpec((blk, D), lambda i: (i, 0))
    w_spec = pl.BlockSpec((D, D), lambda i: (0, 0))
    b_spec = pl.BlockSpec((1, D), lambda i: (0, 0))
    return pl.pallas_call(
        _mlp_body,
        grid=(n // blk,),
        in_specs=[row_spec, w_spec, b_spec, w_spec, b_spec],
        out_specs=row_spec,
        out_shape=jax.ShapeDtypeStruct((n, D), jnp.float32),
    )(h, w1, b1.reshape(1, D), w2, b2.reshape(1, D))


def _emb_body(x_ref, w_ref, b_ref, o_ref):
    o_ref[...] = jnp.dot(x_ref[...], w_ref[...],
                         preferred_element_type=jnp.float32) + b_ref[...]


def _emb(x, w, b):
    n = x.shape[0]
    blk = 2000
    return pl.pallas_call(
        _emb_body,
        grid=(n // blk,),
        in_specs=[pl.BlockSpec((blk, D), lambda i: (i, 0)),
                  pl.BlockSpec((D, D), lambda i: (0, 0)),
                  pl.BlockSpec((1, D), lambda i: (0, 0))],
        out_specs=pl.BlockSpec((blk, D), lambda i: (i, 0)),
        out_shape=jax.ShapeDtypeStruct((n, D), jnp.float32),
    )(x, w, b.reshape(1, D))


def kernel(x_N, W_emb, b_emb, gin_W1, gin_b1, gin_W2, gin_b2, W_pred, b_pred,
           ori_node_idx, node2edge, ori_edge_idx, edge2node,
           edge_index_N, edge_index_E, batch):
    node_x = _emb(x_N, W_emb, b_emb)
    prep_N = _edge_prep(edge_index_N)
    prep_E = _edge_prep(edge_index_E)
    xs = [node_x]
    for l in range(NUM_LAYERS):
        _nx = node_x[ori_node_idx]
        for c in range(INNER_LAYERS):
            idx = l * 4 + c
            h = _sc_agg(_nx, *prep_N)
            _nx = _gin_mlp(h, gin_W1[idx], gin_b1[idx],
                           gin_W2[idx], gin_b2[idx])
        edge_x = jax.nn.relu(jax.ops.segment_sum(_nx, node2edge,
                                                 num_segments=N_HYPEREDGES))
        _ex = edge_x[ori_edge_idx]
        for c in range(INNER_LAYERS):
            idx = l * 4 + 2 + c
            h = _sc_agg(_ex, *prep_E)
            _ex = _gin_mlp(h, gin_W1[idx], gin_b1[idx],
                           gin_W2[idx], gin_b2[idx])
        node_x = jax.nn.relu(jax.ops.segment_sum(_ex, edge2node,
                                                 num_segments=N_NODES))
        xs.append(node_x)
    score = jnp.zeros((NUM_GRAPHS, NUM_CLASSES), jnp.float32)
    for i, x in enumerate(xs):
        pooled = jax.ops.segment_sum(x[ori_node_idx], batch,
                                     num_segments=NUM_GRAPHS)
        score = score + pooled @ W_pred[i] + b_pred[i]
    return score


# speedup vs baseline: 3.4813x; 1.0553x over previous
import functools

import jax
import jax.numpy as jnp
from jax import lax
from jax.experimental import pallas as pl
from jax.experimental.pallas import tpu as pltpu
from jax.experimental.pallas import tpu_sc as plsc

N_NODES = 10000
N_HYPEREDGES = 5000
NNZ = 320000
E_INNER = 640000
D = 128
NUM_CLASSES = 10
NUM_GRAPHS = 16
NUM_LAYERS = 2
INNER_LAYERS = 2

NT = 32
ROWS_PT = NNZ // NT
W = 512
NWIN = -(-ROWS_PT // W)
EB = 128
WIN = EB + 16


def _agg_body(x_hbm, srcs_hbm, dsts_hbm, tbl_hbm, out_hbm,
              outb, offv, widxA, wdstA, rowsA, widxB, wdstB, rowsB,
              semwA, semwB, semgA, semgB):
    cid = lax.axis_index("c")
    sid = lax.axis_index("s")
    wid = cid * 16 + sid
    iota = lax.iota(jnp.int32, 16)

    def fire_windows(b, eb_a, widx, wdst, semw):
        wa = eb_a + b * EB
        pltpu.async_copy(srcs_hbm.at[pl.ds(wa, WIN)], widx, semw)
        pltpu.async_copy(dsts_hbm.at[pl.ds(wa, WIN)], wdst, semw)

    def wait_windows(b, eb_a, widx, wdst, semw):
        wa = eb_a + b * EB
        pltpu.make_async_copy(srcs_hbm.at[pl.ds(wa, WIN)], widx, semw).wait()
        pltpu.make_async_copy(dsts_hbm.at[pl.ds(wa, WIN)], wdst, semw).wait()

    def fire_gather(widx, rows, semg):
        pltpu.async_copy(x_hbm.at[widx.at[pl.ds(0, EB)]],
                         rows.at[pl.ds(0, EB)], semg)
        pltpu.async_copy(x_hbm.at[widx.at[pl.ds(EB, 16)]],
                         rows.at[pl.ds(EB, 16)], semg)

    def wait_gather(widx, rows, semg):
        pltpu.make_async_copy(x_hbm.at[widx.at[pl.ds(0, EB)]],
                              rows.at[pl.ds(0, EB)], semg).wait()
        pltpu.make_async_copy(x_hbm.at[widx.at[pl.ds(EB, 16)]],
                              rows.at[pl.ds(EB, 16)], semg).wait()

    def rmw(b, eb, ee, eb_a, wr0, wdst, rows):
        wa = eb_a + b * EB
        bstart = eb + b * EB
        bend = jnp.minimum(bstart + EB, ee)

        def group(g, carry):
            pos = wa + g * 16 + iota
            dvv = wdst[pl.ds(g * 16, 16)]
            ok = (pos >= bstart) & (pos < bend)
            dvl = jnp.where(ok, dvv - wr0, W)
            for l in range(16):
                dv = dvl[l]
                e = g * 16 + l
                for k in range(D // 16):
                    outb[dv, pl.ds(k * 16, 16)] = (
                        outb[dv, pl.ds(k * 16, 16)]
                        + rows[e, pl.ds(k * 16, 16)])
            return carry

        lax.fori_loop(0, WIN // 16, group, 0)

    def win_body(w, carry):
        pltpu.sync_copy(tbl_hbm.at[wid * NWIN + w], offv)
        v = offv[...]
        eb = v[0]
        ee = v[1]
        wr0 = pl.multiple_of(v[2], 8)
        pltpu.sync_copy(x_hbm.at[pl.ds(wr0, W)], outb.at[pl.ds(0, W)])
        eb_a = pl.multiple_of(lax.div(eb, 8) * 8, 8)
        nb = lax.div(ee - eb + (EB - 1), EB)

        @pl.when(nb > 0)
        def _():
            fire_windows(0, eb_a, widxA, wdstA, semwA)
            wait_windows(0, eb_a, widxA, wdstA, semwA)
            fire_gather(widxA, rowsA, semgA)

            @pl.when(nb > 1)
            def _():
                fire_windows(1, eb_a, widxB, wdstB, semwB)

            def pair_body(i, c2):
                bA = 2 * i
                bB = 2 * i + 1

                @pl.when(bB < nb)
                def _():
                    wait_windows(bB, eb_a, widxB, wdstB, semwB)
                    fire_gather(widxB, rowsB, semgB)
                wait_gather(widxA, rowsA, semgA)
                rmw(bA, eb, ee, eb_a, wr0, wdstA, rowsA)

                @pl.when(bB + 1 < nb)
                def _():
                    fire_windows(bB + 1, eb_a, widxA, wdstA, semwA)

                @pl.when(bB < nb)
                def _():
                    wait_gather(widxB, rowsB, semgB)
                    rmw(bB, eb, ee, eb_a, wr0, wdstB, rowsB)

                @pl.when(bB + 1 < nb)
                def _():
                    wait_windows(bB + 1, eb_a, widxA, wdstA, semwA)
                    fire_gather(widxA, rowsA, semgA)

                @pl.when(bB + 2 < nb)
                def _():
                    fire_windows(bB + 2, eb_a, widxB, wdstB, semwB)
                return c2

            lax.fori_loop(0, lax.div(nb + 1, 2), pair_body, 0)

        pltpu.sync_copy(outb.at[pl.ds(0, W)], out_hbm.at[pl.ds(wr0, W)])
        return carry

    lax.fori_loop(0, NWIN, win_body, 0)


_sc_agg = pl.kernel(
    _agg_body,
    out_type=jax.ShapeDtypeStruct((NNZ, D), jnp.float32),
    mesh=plsc.VectorSubcoreMesh(core_axis_name="c", subcore_axis_name="s"),
    scratch_types=[
        pltpu.VMEM((W + 8, D), jnp.float32),
        pltpu.VMEM((16,), jnp.int32),
        pltpu.VMEM((WIN,), jnp.int32),
        pltpu.VMEM((WIN,), jnp.int32),
        pltpu.VMEM((WIN, D), jnp.float32),
        pltpu.VMEM((WIN,), jnp.int32),
        pltpu.VMEM((WIN,), jnp.int32),
        pltpu.VMEM((WIN, D), jnp.float32),
        pltpu.SemaphoreType.DMA,
        pltpu.SemaphoreType.DMA,
        pltpu.SemaphoreType.DMA,
        pltpu.SemaphoreType.DMA,
    ],
)


def _edge_prep(edge_index):
    src, dst = edge_index[0], edge_index[1]
    order = jnp.argsort(dst)
    src_s = jnp.take(src, order)
    dst_s = jnp.take(dst, order)
    wr0_rel = jnp.minimum(jnp.arange(NWIN, dtype=jnp.int32) * W, ROWS_PT - W)
    wr0 = (jnp.arange(NT, dtype=jnp.int32)[:, None] * ROWS_PT
           + wr0_rel[None, :]).reshape(-1)
    bnds = jnp.stack([wr0, wr0 + W], 1).reshape(-1)
    ebee = jnp.searchsorted(dst_s, bnds).astype(jnp.int32).reshape(-1, 2)
    tbl = jnp.concatenate(
        [ebee, wr0[:, None], jnp.zeros((NT * NWIN, 1), jnp.int32)], axis=1)
    tbl16 = jnp.tile(tbl, (1, 4))
    src_p = jnp.concatenate([src_s, jnp.zeros((2 * WIN + 8,), jnp.int32)])
    dst_p = jnp.concatenate([dst_s, jnp.full((2 * WIN + 8,), NNZ, jnp.int32)])
    return src_p, dst_p, tbl16


def _make_segsum(S, SW):
    def body(x_hbm, segp_hbm, tbl_hbm, zeros_hbm, out_hbm,
             outb, offv, wsegA, rowsA, wsegB, rowsB, semA, semB):
        cid = lax.axis_index("c")
        sid = lax.axis_index("s")
        wid = cid * 16 + sid
        iota = lax.iota(jnp.int32, 16)
        pltpu.sync_copy(tbl_hbm.at[wid], offv)
        v = offv[...]
        eb = v[0]
        ee = v[1]
        r0 = pl.multiple_of(v[2], 8)
        pltpu.sync_copy(zeros_hbm, outb.at[pl.ds(0, SW)])
        eb_a = pl.multiple_of(lax.div(eb, 8) * 8, 8)
        nb = lax.div(ee - eb + (EB - 1), EB)

        def wa_of(b):
            return pl.multiple_of(
                jnp.minimum(eb_a + b * EB, NNZ - WIN), 8)

        def fire(b, wseg, rows, sem):
            wa = wa_of(b)
            pltpu.async_copy(x_hbm.at[pl.ds(wa, WIN)], rows, sem)
            pltpu.async_copy(segp_hbm.at[pl.ds(wa, WIN)], wseg, sem)

        def wait(b, wseg, rows, sem):
            wa = wa_of(b)
            pltpu.make_async_copy(x_hbm.at[pl.ds(wa, WIN)], rows, sem).wait()
            pltpu.make_async_copy(segp_hbm.at[pl.ds(wa, WIN)], wseg, sem).wait()

        def rmw(b, wseg, rows):
            wa = wa_of(b)
            bstart = eb + b * EB
            bend = jnp.minimum(bstart + EB, ee)

            def group(g, carry):
                pos = wa + g * 16 + iota
                dvv = wseg[pl.ds(g * 16, 16)]
                ok = (pos >= bstart) & (pos < bend)
                dvl = jnp.where(ok, dvv - r0, SW)
                for l in range(16):
                    dv = dvl[l]
                    e = g * 16 + l
                    for k in range(D // 16):
                        outb[dv, pl.ds(k * 16, 16)] = (
                            outb[dv, pl.ds(k * 16, 16)]
                            + rows[e, pl.ds(k * 16, 16)])
                return carry

            lax.fori_loop(0, WIN // 16, group, 0)

        @pl.when(nb > 0)
        def _():
            fire(0, wsegA, rowsA, semA)

            @pl.when(nb > 1)
            def _():
                fire(1, wsegB, rowsB, semB)

            def pair_body(i, c2):
                bA = 2 * i
                bB = 2 * i + 1
                wait(bA, wsegA, rowsA, semA)
                rmw(bA, wsegA, rowsA)

                @pl.when(bB + 1 < nb)
                def _():
                    fire(bB + 1, wsegA, rowsA, semA)

                @pl.when(bB < nb)
                def _():
                    wait(bB, wsegB, rowsB, semB)
                    rmw(bB, wsegB, rowsB)

                @pl.when(bB + 2 < nb)
                def _():
                    fire(bB + 2, wsegB, rowsB, semB)
                return c2

            lax.fori_loop(0, lax.div(nb + 1, 2), pair_body, 0)

        def relu_row(r, carry):
            for k in range(D // 16):
                outb[r, pl.ds(k * 16, 16)] = jnp.maximum(
                    outb[r, pl.ds(k * 16, 16)], 0.0)
            return carry

        lax.fori_loop(0, SW, relu_row, 0)
        pltpu.sync_copy(outb.at[pl.ds(0, SW)], out_hbm.at[pl.ds(r0, SW)])

    return pl.kernel(
        body,
        out_type=jax.ShapeDtypeStruct((S, D), jnp.float32),
        mesh=plsc.VectorSubcoreMesh(core_axis_name="c", subcore_axis_name="s"),
        scratch_types=[
            pltpu.VMEM((SW + 8, D), jnp.float32),
            pltpu.VMEM((16,), jnp.int32),
            pltpu.VMEM((WIN,), jnp.int32),
            pltpu.VMEM((WIN, D), jnp.float32),
            pltpu.VMEM((WIN,), jnp.int32),
            pltpu.VMEM((WIN, D), jnp.float32),
            pltpu.SemaphoreType.DMA,
            pltpu.SemaphoreType.DMA,
        ],
    )


_sc_segsum_edge = _make_segsum(N_HYPEREDGES, 160)
_sc_segsum_node = _make_segsum(N_NODES, 320)


def _seg_prep(seg, S, SW):
    r0 = jnp.minimum(jnp.arange(NT, dtype=jnp.int32) * SW, S - SW)
    bnds = jnp.stack([r0, r0 + SW], 1).reshape(-1)
    ebee = jnp.searchsorted(seg, bnds).astype(jnp.int32).reshape(-1, 2)
    tbl16 = jnp.tile(jnp.concatenate(
        [ebee, r0[:, None], jnp.zeros((NT, 1), jnp.int32)], axis=1), (1, 4))
    segp = jnp.concatenate([seg, jnp.full((2 * WIN + 8,), S, jnp.int32)])
    return segp, tbl16


def _mlp_body(h_ref, w1_ref, b1_ref, w2_ref, b2_ref, o_ref):
    h = jnp.maximum(jnp.dot(h_ref[...], w1_ref[...],
                            preferred_element_type=jnp.float32)
                    + b1_ref[...], 0.0)
    o_ref[...] = jnp.maximum(jnp.dot(h, w2_ref[...],
                                     preferred_element_type=jnp.float32)
                             + b2_ref[...], 0.0)


def _gin_mlp(h, w1, b1, w2, b2):
    n = h.shape[0]
    blk = 2000
    row_spec = pl.BlockSpec((blk, D), lambda i: (i, 0))
    w_spec = pl.BlockSpec((D, D), lambda i: (0, 0))
    b_spec = pl.BlockSpec((1, D), lambda i: (0, 0))
    return pl.pallas_call(
        _mlp_body,
        grid=(n // blk,),
        in_specs=[row_spec, w_spec, b_spec, w_spec, b_spec],
        out_specs=row_spec,
        out_shape=jax.ShapeDtypeStruct((n, D), jnp.float32),
    )(h, w1, b1.reshape(1, D), w2, b2.reshape(1, D))


def _emb_body(x_ref, w_ref, b_ref, o_ref):
    o_ref[...] = jnp.dot(x_ref[...], w_ref[...],
                         preferred_element_type=jnp.float32) + b_ref[...]


def _emb(x, w, b):
    n = x.shape[0]
    blk = 2000
    return pl.pallas_call(
        _emb_body,
        grid=(n // blk,),
        in_specs=[pl.BlockSpec((blk, D), lambda i: (i, 0)),
                  pl.BlockSpec((D, D), lambda i: (0, 0)),
                  pl.BlockSpec((1, D), lambda i: (0, 0))],
        out_specs=pl.BlockSpec((blk, D), lambda i: (i, 0)),
        out_shape=jax.ShapeDtypeStruct((n, D), jnp.float32),
    )(x, w, b.reshape(1, D))


def kernel(x_N, W_emb, b_emb, gin_W1, gin_b1, gin_W2, gin_b2, W_pred, b_pred,
           ori_node_idx, node2edge, ori_edge_idx, edge2node,
           edge_index_N, edge_index_E, batch):
    node_x = _emb(x_N, W_emb, b_emb)
    prep_N = _edge_prep(edge_index_N)
    prep_E = _edge_prep(edge_index_E)
    segp_n2e, tbl_n2e = _seg_prep(node2edge, N_HYPEREDGES, 160)
    segp_e2n, tbl_e2n = _seg_prep(edge2node, N_NODES, 320)
    zeros_e = jnp.zeros((160, D), jnp.float32)
    zeros_n = jnp.zeros((320, D), jnp.float32)
    xs = [node_x]
    for l in range(NUM_LAYERS):
        _nx = node_x[ori_node_idx]
        for c in range(INNER_LAYERS):
            idx = l * 4 + c
            h = _sc_agg(_nx, *prep_N)
            _nx = _gin_mlp(h, gin_W1[idx], gin_b1[idx],
                           gin_W2[idx], gin_b2[idx])
        edge_x = _sc_segsum_edge(_nx, segp_n2e, tbl_n2e, zeros_e)
        _ex = edge_x[ori_edge_idx]
        for c in range(INNER_LAYERS):
            idx = l * 4 + 2 + c
            h = _sc_agg(_ex, *prep_E)
            _ex = _gin_mlp(h, gin_W1[idx], gin_b1[idx],
                           gin_W2[idx], gin_b2[idx])
        node_x = _sc_segsum_node(_ex, segp_e2n, tbl_e2n, zeros_n)
        xs.append(node_x)
    score = jnp.zeros((NUM_GRAPHS, NUM_CLASSES), jnp.float32)
    for i, x in enumerate(xs):
        pooled = jax.ops.segment_sum(x[ori_node_idx], batch,
                                     num_segments=NUM_GRAPHS)
        score = score + pooled @ W_pred[i] + b_pred[i]
    return score
